# baseline (speedup 1.0000x reference)
import functools

import jax
import jax.numpy as jnp
from jax import lax
from jax.experimental import pallas as pl
from jax.experimental.pallas import tpu as pltpu
from jax.experimental.pallas import tpu_sc as plsc

N = 10000
E = 320000
D = 128
DE = 16

NC, NS, L = 2, 16, 16
NW = NC * NS
NPW = 313
NPAD = NW * NPW
EPW = E // NW
ECAP = E + 2560
SLOT_BITS = 9

_MESH = plsc.VectorSubcoreMesh(
    core_axis_name="c", subcore_axis_name="s", num_cores=NC, num_subcores=NS)


def _wid():
    return lax.axis_index("s") * NC + lax.axis_index("c")


_BK_CHUNK = 8000
_BK_U = 4
_BK_SUB = 25
_BK_NSUB = _BK_CHUNK // (L * _BK_SUB * _BK_U)
_BK_FLUSH = 8192
_BK_CAP = _BK_FLUSH + _BK_SUB * _BK_U * L + 32


def _bucketize_body(dst_hbm, lists_hbm, counts_hbm, dstbuf, buf, cnt_v):
    w = _wid()
    lo = w * NPW
    lane = jax.lax.iota(jnp.int32, L)
    zero16 = jnp.zeros((L,), jnp.int32)

    def chunk_body(c, carry):
        fill_v, off = carry
        base = c * _BK_CHUNK
        pltpu.sync_copy(dst_hbm.at[pl.ds(pl.multiple_of(base, 16), _BK_CHUNK)], dstbuf)

        def sub_block(sb, carry):
            fill_v, off = carry

            def vec_body(i, fill_v):
                k0 = (sb * _BK_SUB + i) * _BK_U
                dvs = [dstbuf[pl.ds((k0 + u) * L, L)] for u in range(_BK_U)]
                for u in range(_BK_U):
                    d = dvs[u]
                    ids = jnp.full((L,), base + (k0 + u) * L, jnp.int32) + lane
                    slot = d - lo
                    m = (slot >= 0) & (slot < NPW)
                    word = (ids << SLOT_BITS) | slot
                    pos = fill_v + plsc.cumsum(m.astype(jnp.int32)) - 1
                    plsc.store_scatter(buf, [pos], word, mask=m)
                    fill_v = fill_v + plsc.all_reduce_population_count(m)
                return fill_v

            fill_v = lax.fori_loop(0, _BK_SUB, vec_body, fill_v)
            fillmax = jnp.max(fill_v)

            def do_flush(fill_v, off):
                pltpu.sync_copy(buf.at[pl.ds(0, _BK_FLUSH)],
                                lists_hbm.at[pl.ds(pl.multiple_of(w * ECAP + off, 128), _BK_FLUSH)])
                for j in range((_BK_CAP - _BK_FLUSH) // L):
                    buf[pl.ds(j * L, L)] = buf[pl.ds(_BK_FLUSH + j * L, L)]
                return fill_v - _BK_FLUSH, off + _BK_FLUSH

            return lax.cond(fillmax >= _BK_FLUSH, do_flush,
                            lambda f, o: (f, o), fill_v, off)

        return lax.fori_loop(0, _BK_NSUB, sub_block, (fill_v, off))

    fill_v, off = lax.fori_loop(0, E // _BK_CHUNK, chunk_body, (zero16, 0))
    fill = jnp.max(fill_v)

    def pad_and_flush(fill, off):
        last_v = buf[pl.ds(fill - 1, L)]
        pad_word = jnp.full((L,), last_v[0], jnp.int32)
        for j in range(128 // L):
            buf[pl.ds(fill + j * L, L)] = pad_word
        padded = ((fill + 127) // 128) * 128

        def fb(b, off):
            pltpu.sync_copy(buf.at[pl.ds(b * 128, 128)],
                            lists_hbm.at[pl.ds(pl.multiple_of(w * ECAP + off + b * 128, 128), 128)])
            return off
        lax.fori_loop(0, padded // 128, fb, off)
        return off + padded

    total = lax.cond(fill > 0, pad_and_flush, lambda f, o: o, fill, off)
    for j in range(128 // L):
        cnt_v[pl.ds(j * L, L)] = jnp.full((L,), total, jnp.int32)
    pltpu.sync_copy(cnt_v, counts_hbm.at[pl.ds(pl.multiple_of(w * 128, 128), 128)])


_bucketize = functools.partial(
    pl.kernel, _bucketize_body,
    out_type=(jax.ShapeDtypeStruct((NW * ECAP,), jnp.int32),
              jax.ShapeDtypeStruct((NW * 128,), jnp.int32)),
    mesh=_MESH,
    compiler_params=pltpu.CompilerParams(needs_layout_passes=False),
    scratch_types=[pltpu.VMEM((_BK_CHUNK,), jnp.int32),
                   pltpu.VMEM((_BK_CAP,), jnp.int32),
                   pltpu.VMEM((128,), jnp.int32)],
)()


_GB = 80


_NGB = EPW // _GB


def _gather_pipe_body(do_add, d, ta_hbm, tb_hbm, dst_hbm, src_hbm,
                      oa_hbm, ob_hbm, idxd, idxs,
                      rA0, rB0, rA1, rB1, semG0, semG1, semW0, semW1):
    w = _wid()
    base = pl.multiple_of(w * EPW, 16)
    pltpu.sync_copy(dst_hbm.at[pl.ds(base, EPW)], idxd)
    pltpu.sync_copy(src_hbm.at[pl.ds(base, EPW)], idxs)

    def stage(b, rA, rB, semG):
        s = pl.ds(b * _GB, _GB)
        pltpu.async_copy(ta_hbm.at[idxd.at[s]], rA, semG)
        pltpu.async_copy(tb_hbm.at[idxs.at[s]], rB, semG)

    def wait_g(rA, rB, semG):
        pltpu.make_async_copy(ta_hbm.at[idxd.at[pl.ds(0, _GB)]], rA, semG).wait()
        pltpu.make_async_copy(tb_hbm.at[idxs.at[pl.ds(0, _GB)]], rB, semG).wait()

    def fire_w(b, rA, rB, semW):
        off = pl.multiple_of(w * EPW + b * _GB, 16)
        if do_add:
            def add_row(r, _):
                va = [rA[r, pl.ds(v * L, L)] for v in range(d // L)]
                vb = [rB[r, pl.ds(v * L, L)] for v in range(d // L)]
                for v in range(d // L):
                    rA[r, pl.ds(v * L, L)] = va[v] + vb[v]
                return 0
            lax.fori_loop(0, _GB, add_row, 0)
            pltpu.async_copy(rA, oa_hbm.at[pl.ds(off, _GB)], semW)
        else:
            pltpu.async_copy(rA, oa_hbm.at[pl.ds(off, _GB)], semW)
            pltpu.async_copy(rB, ob_hbm.at[pl.ds(off, _GB)], semW)

    def wait_w(rA, rB, semW):
        pltpu.make_async_copy(rA, oa_hbm.at[pl.ds(0, _GB)], semW).wait()
        if not do_add:
            pltpu.make_async_copy(rB, ob_hbm.at[pl.ds(0, _GB)], semW).wait()

    stage(0, rA0, rB0, semG0)

    def pair(p, _):
        b0 = 2 * p
        b1 = 2 * p + 1

        @pl.when(b1 < _NGB)
        def _():
            @pl.when(p > 0)
            def _():
                wait_w(rA1, rB1, semW1)
            stage(b1, rA1, rB1, semG1)

        wait_g(rA0, rB0, semG0)
        fire_w(b0, rA0, rB0, semW0)

        @pl.when(b0 + 2 < _NGB)
        def _():
            wait_w(rA0, rB0, semW0)
            stage(b0 + 2, rA0, rB0, semG0)

        @pl.when(b1 < _NGB)
        def _():
            wait_g(rA1, rB1, semG1)
            fire_w(b1, rA1, rB1, semW1)
        return 0

    lax.fori_loop(0, (_NGB + 1) // 2, pair, 0)
    wait_w(rA0, rB0, semW0)
    if _NGB > 1:
        wait_w(rA1, rB1, semW1)


def _gather_scratch(d):
    return [pltpu.VMEM((EPW,), jnp.int32),
            pltpu.VMEM((EPW,), jnp.int32),
            pltpu.VMEM((_GB, d), jnp.float32),
            pltpu.VMEM((_GB, d), jnp.float32),
            pltpu.VMEM((_GB, d), jnp.float32),
            pltpu.VMEM((_GB, d), jnp.float32),
            pltpu.SemaphoreType.DMA,
            pltpu.SemaphoreType.DMA,
            pltpu.SemaphoreType.DMA,
            pltpu.SemaphoreType.DMA]


def _g2_body(table_hbm, dst_hbm, src_hbm, xi_hbm, xj_hbm, *rest):
    _gather_pipe_body(False, D, table_hbm, table_hbm, dst_hbm, src_hbm,
                      xi_hbm, xj_hbm, *rest)


_gather2 = functools.partial(
    pl.kernel, _g2_body,
    out_type=(jax.ShapeDtypeStruct((E, D), jnp.float32),
              jax.ShapeDtypeStruct((E, D), jnp.float32)),
    mesh=_MESH,
    compiler_params=pltpu.CompilerParams(needs_layout_passes=False),
    scratch_types=_gather_scratch(D),
)()


_D2 = 256


def _ga_body(ta_hbm, tb_hbm, dst_hbm, src_hbm, g_hbm, *rest):
    _gather_pipe_body(True, _D2, ta_hbm, tb_hbm, dst_hbm, src_hbm,
                      g_hbm, g_hbm, *rest)


_gather_add = functools.partial(
    pl.kernel, _ga_body,
    out_type=jax.ShapeDtypeStruct((E, _D2), jnp.float32),
    mesh=_MESH,
    compiler_params=pltpu.CompilerParams(needs_layout_passes=False),
    scratch_types=_gather_scratch(_D2),
)()


_SGB = {256: 64, 128: 128}


_WCH = 2048


def _segmax_body(dm, m_hbm, lists_hbm, counts_hbm, h_hbm,
                 wbig, idb0, idb1, slb0, slb1, rows0, rows1,
                 acc, cnt_v, sem0, sem1):
    w = _wid()
    bg = _SGB[dm]
    wpb = _WCH // bg

    def init_f(r, _):
        acc[pl.ds(r * L, L)] = jnp.zeros((L,), jnp.float32)
        return 0
    lax.fori_loop(0, NPW * dm // L, init_f, 0)

    pltpu.sync_copy(counts_hbm.at[pl.ds(pl.multiple_of(w * 128, 128), 128)], cnt_v)
    cnt = cnt_v[pl.ds(0, L)][0]
    nb = cnt // bg

    def stage(b, idb, slb, rows, sem):
        @pl.when(b % wpb == 0)
        def _():
            pltpu.sync_copy(
                lists_hbm.at[pl.ds(pl.multiple_of(w * ECAP, 128)
                                   + (b // wpb) * _WCH, _WCH)],
                wbig)
        rel = (b % wpb) * bg
        for j8 in range(bg // L):
            s = pl.ds(j8 * L, L)
            wv = wbig[pl.ds(rel + j8 * L, L)]
            idb[s] = wv >> SLOT_BITS
            slb[s] = wv & ((1 << SLOT_BITS) - 1)
        pltpu.async_copy(m_hbm.at[idb], rows, sem)

    def consume(slb, rows):
        def upd(j8, _):
            sv = slb[pl.ds(j8 * L, L)] * dm
            bases = [sv[jj] for jj in range(L)]
            for jj in range(L):
                j = j8 * L + jj
                base = bases[jj]
                rvals = [rows[j, pl.ds(v * L, L)] for v in range(dm // L)]
                avals = [acc[pl.ds(base + v * L, L)] for v in range(dm // L)]
                for v in range(dm // L):
                    acc[pl.ds(base + v * L, L)] = jnp.maximum(avals[v], rvals[v])
            return 0
        lax.fori_loop(0, bg // L, upd, 0)

    @pl.when(nb > 0)
    def _():
        stage(0, idb0, slb0, rows0, sem0)

    def pair(p, _):
        b0 = 2 * p
        b1 = 2 * p + 1

        @pl.when(b1 < nb)
        def _():
            stage(b1, idb1, slb1, rows1, sem1)
        pltpu.make_async_copy(m_hbm.at[idb0], rows0, sem0).wait()
        consume(slb0, rows0)

        @pl.when(b0 + 2 < nb)
        def _():
            stage(b0 + 2, idb0, slb0, rows0, sem0)

        @pl.when(b1 < nb)
        def _():
            pltpu.make_async_copy(m_hbm.at[idb1], rows1, sem1).wait()
            consume(slb1, rows1)
        return 0

    lax.fori_loop(0, (nb + 1) // 2, pair, 0)
    pltpu.sync_copy(acc, h_hbm.at[pl.ds(pl.multiple_of(w * NPW * dm, 128), NPW * dm)])


def _make_segmax(dm):
    bg = _SGB[dm]
    return functools.partial(
        pl.kernel, functools.partial(_segmax_body, dm),
        out_type=jax.ShapeDtypeStruct((NPAD * dm,), jnp.float32),
        mesh=_MESH,
        compiler_params=pltpu.CompilerParams(needs_layout_passes=False),
        scratch_types=[pltpu.VMEM((_WCH,), jnp.int32),
                       pltpu.VMEM((bg,), jnp.int32),
                       pltpu.VMEM((bg,), jnp.int32),
                       pltpu.VMEM((bg,), jnp.int32),
                       pltpu.VMEM((bg,), jnp.int32),
                       pltpu.VMEM((bg, dm), jnp.float32),
                       pltpu.VMEM((bg, dm), jnp.float32),
                       pltpu.VMEM((NPW * dm,), jnp.float32),
                       pltpu.VMEM((128,), jnp.int32),
                       pltpu.SemaphoreType.DMA,
                       pltpu.SemaphoreType.DMA],
    )()


_segmax_256 = _make_segmax(256)
_segmax_128 = _make_segmax(128)


_BE = 512


def _mlp1_block(xi, xj, ea, wd, wm, we, b1a, w1b, b1b, out):
    h = (jnp.dot(xi[...].astype(jnp.bfloat16), wd[...],
                 preferred_element_type=jnp.float32)
         + jnp.dot(xj[...].astype(jnp.bfloat16), wm[...],
                   preferred_element_type=jnp.float32)
         + jnp.dot(ea[...], we[...], preferred_element_type=jnp.float32)
         + b1a[...])
    h = jnp.maximum(h, 0.0).astype(jnp.bfloat16)
    h2 = jnp.dot(h, w1b[...], preferred_element_type=jnp.float32) + b1b[...]
    out[...] = jnp.maximum(h2, 0.0)


def _mlp1(xi, xj, ea, wd, wm, we, b1a, w1b, b1b):
    grid = (E // _BE,)
    return pl.pallas_call(
        _mlp1_block,
        grid=grid,
        in_specs=[
            pl.BlockSpec((_BE, D), lambda i: (i, 0)),
            pl.BlockSpec((_BE, D), lambda i: (i, 0)),
            pl.BlockSpec((_BE, DE), lambda i: (i, 0)),
            pl.BlockSpec((D, 512), lambda i: (0, 0)),
            pl.BlockSpec((D, 512), lambda i: (0, 0)),
            pl.BlockSpec((DE, 512), lambda i: (0, 0)),
            pl.BlockSpec((1, 512), lambda i: (0, 0)),
            pl.BlockSpec((512, 256), lambda i: (0, 0)),
            pl.BlockSpec((1, 256), lambda i: (0, 0)),
        ],
        out_specs=pl.BlockSpec((_BE, 256), lambda i: (i, 0)),
        out_shape=jax.ShapeDtypeStruct((E, 256), jnp.float32),
    )(xi, xj, ea, wd, wm, we, b1a, w1b, b1b)


def _pre2_block(h1, wd, wm, a2, b2):
    hb = h1[...].astype(jnp.bfloat16)
    a2[...] = jnp.dot(hb, wd[...], preferred_element_type=jnp.float32)
    b2[...] = jnp.dot(hb, wm[...], preferred_element_type=jnp.float32)


def _pre2(h1, wd, wm):
    bn = NPAD // 4
    return pl.pallas_call(
        _pre2_block,
        grid=(4,),
        in_specs=[pl.BlockSpec((bn, _D2), lambda i: (i, 0)),
                  pl.BlockSpec((_D2, _D2), lambda i: (0, 0)),
                  pl.BlockSpec((_D2, _D2), lambda i: (0, 0))],
        out_specs=(pl.BlockSpec((bn, _D2), lambda i: (i, 0)),
                   pl.BlockSpec((bn, _D2), lambda i: (i, 0))),
        out_shape=(jax.ShapeDtypeStruct((NPAD, _D2), jnp.float32),
                   jax.ShapeDtypeStruct((NPAD, _D2), jnp.float32)),
    )(h1, wd, wm)


def _mlp2_block(g2, ea, we, b2a, w2b, b2b, out):
    h = (g2[...]
         + jnp.dot(ea[...], we[...], preferred_element_type=jnp.float32)
         + b2a[...])
    h = jnp.maximum(h, 0.0).astype(jnp.bfloat16)
    h2 = jnp.dot(h, w2b[...], preferred_element_type=jnp.float32) + b2b[...]
    out[...] = jnp.maximum(h2, 0.0)


def _mlp2(g2, ea, we, b2a, w2b, b2b):
    return pl.pallas_call(
        _mlp2_block,
        grid=(E // _BE,),
        in_specs=[
            pl.BlockSpec((_BE, _D2), lambda i: (i, 0)),
            pl.BlockSpec((_BE, DE), lambda i: (i, 0)),
            pl.BlockSpec((DE, _D2), lambda i: (0, 0)),
            pl.BlockSpec((1, _D2), lambda i: (0, 0)),
            pl.BlockSpec((_D2, D), lambda i: (0, 0)),
            pl.BlockSpec((1, D), lambda i: (0, 0)),
        ],
        out_specs=pl.BlockSpec((_BE, D), lambda i: (i, 0)),
        out_shape=jax.ShapeDtypeStruct((E, D), jnp.float32),
    )(g2, ea, we, b2a, w2b, b2b)


def _final_block(h2, w3, b3, w4, b4, out):
    h = jnp.dot(h2[...], w3[...], preferred_element_type=jnp.float32) + b3[...]
    h = jnp.maximum(h, 0.0)
    o = jnp.dot(h, w4[...], preferred_element_type=jnp.float32) + b4[...]
    out[...] = jax.nn.sigmoid(o)


def _final(h2, w3, b3, w4, b4):
    bn = NPAD // 4
    return pl.pallas_call(
        _final_block,
        grid=(4,),
        in_specs=[
            pl.BlockSpec((bn, D), lambda i: (i, 0)),
            pl.BlockSpec((D, 64), lambda i: (0, 0)),
            pl.BlockSpec((1, 64), lambda i: (0, 0)),
            pl.BlockSpec((64, 1), lambda i: (0, 0)),
            pl.BlockSpec((1, 1), lambda i: (0, 0)),
        ],
        out_specs=pl.BlockSpec((bn, 1), lambda i: (i, 0)),
        out_shape=jax.ShapeDtypeStruct((NPAD, 1), jnp.float32),
    )(h2, w3, b3, w4, b4)


def kernel(x, edge_index, edge_attr, W1a, b1a, W1b, b1b, W2a, b2a, W2b, b2b,
           W3, b3, W4, b4):
    src = edge_index[0].astype(jnp.int32)
    dst = edge_index[1].astype(jnp.int32)
    bf = jnp.bfloat16

    lists, counts = _bucketize(dst)

    xi, xj = _gather2(x, dst, src)
    w1d = (W1a[0:D] - W1a[D:2 * D]).astype(bf)
    m1 = _mlp1(xi, xj, edge_attr.astype(bf),
               w1d, W1a[D:2 * D].astype(bf), W1a[2 * D:].astype(bf),
               b1a.reshape(1, -1), W1b.astype(bf), b1b.reshape(1, -1))
    h1 = _segmax_256(m1, lists, counts).reshape(NPAD, 256)

    w2d = (W2a[0:_D2] - W2a[_D2:2 * _D2]).astype(bf)
    a2, b2t = _pre2(h1, w2d, W2a[_D2:2 * _D2].astype(bf))
    g2 = _gather_add(a2, b2t, dst, src)
    m2 = _mlp2(g2, edge_attr.astype(bf),
               W2a[2 * _D2:].astype(bf), b2a.reshape(1, -1),
               W2b.astype(bf), b2b.reshape(1, -1))
    h2 = _segmax_128(m2, lists, counts).reshape(NPAD, D)

    out = _final(h2, W3, b3.reshape(1, -1), W4, b4.reshape(1, -1))
    return out[:N]


# speedup vs baseline: 1.0720x; 1.0720x over previous
import functools

import jax
import jax.numpy as jnp
from jax import lax
from jax.experimental import pallas as pl
from jax.experimental.pallas import tpu as pltpu
from jax.experimental.pallas import tpu_sc as plsc

N = 10000
E = 320000
D = 128
DE = 16

NC, NS, L = 2, 16, 16
NW = NC * NS
NPW = 313
NPAD = NW * NPW
EPW = E // NW
ECAP = E + 2560
SLOT_BITS = 9

_MESH = plsc.VectorSubcoreMesh(
    core_axis_name="c", subcore_axis_name="s", num_cores=NC, num_subcores=NS)


def _wid():
    return lax.axis_index("s") * NC + lax.axis_index("c")


_BK_CHUNK = 8000
_BK_U = 4
_BK_SUB = 25
_BK_NSUB = _BK_CHUNK // (L * _BK_SUB * _BK_U)
_BK_FLUSH = 8192
_BK_CAP = _BK_FLUSH + _BK_SUB * _BK_U * L + 32


def _bucketize_body(dst_hbm, lists_hbm, counts_hbm, dstbuf, buf, cnt_v):
    w = _wid()
    lo = w * NPW
    lane = jax.lax.iota(jnp.int32, L)
    zero16 = jnp.zeros((L,), jnp.int32)

    def chunk_body(c, carry):
        fill_v, off = carry
        base = c * _BK_CHUNK
        pltpu.sync_copy(dst_hbm.at[pl.ds(pl.multiple_of(base, 16), _BK_CHUNK)], dstbuf)

        def sub_block(sb, carry):
            fill_v, off = carry

            def vec_body(i, fill_v):
                k0 = (sb * _BK_SUB + i) * _BK_U
                dvs = [dstbuf[pl.ds((k0 + u) * L, L)] for u in range(_BK_U)]
                for u in range(_BK_U):
                    d = dvs[u]
                    ids = jnp.full((L,), base + (k0 + u) * L, jnp.int32) + lane
                    slot = d - lo
                    m = (slot >= 0) & (slot < NPW)
                    word = (ids << SLOT_BITS) | slot
                    pos = fill_v + plsc.cumsum(m.astype(jnp.int32)) - 1
                    plsc.store_scatter(buf, [pos], word, mask=m)
                    fill_v = fill_v + plsc.all_reduce_population_count(m)
                return fill_v

            fill_v = lax.fori_loop(0, _BK_SUB, vec_body, fill_v)
            fillmax = jnp.max(fill_v)

            def do_flush(fill_v, off):
                pltpu.sync_copy(buf.at[pl.ds(0, _BK_FLUSH)],
                                lists_hbm.at[pl.ds(pl.multiple_of(w * ECAP + off, 128), _BK_FLUSH)])
                for j in range((_BK_CAP - _BK_FLUSH) // L):
                    buf[pl.ds(j * L, L)] = buf[pl.ds(_BK_FLUSH + j * L, L)]
                return fill_v - _BK_FLUSH, off + _BK_FLUSH

            return lax.cond(fillmax >= _BK_FLUSH, do_flush,
                            lambda f, o: (f, o), fill_v, off)

        return lax.fori_loop(0, _BK_NSUB, sub_block, (fill_v, off))

    fill_v, off = lax.fori_loop(0, E // _BK_CHUNK, chunk_body, (zero16, 0))
    fill = jnp.max(fill_v)

    def pad_and_flush(fill, off):
        last_v = buf[pl.ds(fill - 1, L)]
        pad_word = jnp.full((L,), last_v[0], jnp.int32)
        for j in range(128 // L):
            buf[pl.ds(fill + j * L, L)] = pad_word
        padded = ((fill + 127) // 128) * 128

        def fb(b, off):
            pltpu.sync_copy(buf.at[pl.ds(b * 128, 128)],
                            lists_hbm.at[pl.ds(pl.multiple_of(w * ECAP + off + b * 128, 128), 128)])
            return off
        lax.fori_loop(0, padded // 128, fb, off)
        return off + padded

    total = lax.cond(fill > 0, pad_and_flush, lambda f, o: o, fill, off)
    for j in range(128 // L):
        cnt_v[pl.ds(j * L, L)] = jnp.full((L,), total, jnp.int32)
    pltpu.sync_copy(cnt_v, counts_hbm.at[pl.ds(pl.multiple_of(w * 128, 128), 128)])


_bucketize = functools.partial(
    pl.kernel, _bucketize_body,
    out_type=(jax.ShapeDtypeStruct((NW * ECAP,), jnp.int32),
              jax.ShapeDtypeStruct((NW * 128,), jnp.int32)),
    mesh=_MESH,
    compiler_params=pltpu.CompilerParams(needs_layout_passes=False),
    scratch_types=[pltpu.VMEM((_BK_CHUNK,), jnp.int32),
                   pltpu.VMEM((_BK_CAP,), jnp.int32),
                   pltpu.VMEM((128,), jnp.int32)],
)()


_GB = 80


_NGB = EPW // _GB


def _gather_pipe_body(do_add, d, ta_hbm, tb_hbm, dst_hbm, src_hbm,
                      oa_hbm, ob_hbm, idxd, idxs,
                      rA0, rB0, rA1, rB1, semG0, semG1, semW0, semW1):
    w = _wid()
    base = pl.multiple_of(w * EPW, 16)
    pltpu.sync_copy(dst_hbm.at[pl.ds(base, EPW)], idxd)
    pltpu.sync_copy(src_hbm.at[pl.ds(base, EPW)], idxs)

    def stage(b, rA, rB, semG):
        s = pl.ds(b * _GB, _GB)
        pltpu.async_copy(ta_hbm.at[idxd.at[s]], rA, semG)
        pltpu.async_copy(tb_hbm.at[idxs.at[s]], rB, semG)

    def wait_g(rA, rB, semG):
        pltpu.make_async_copy(ta_hbm.at[idxd.at[pl.ds(0, _GB)]], rA, semG).wait()
        pltpu.make_async_copy(tb_hbm.at[idxs.at[pl.ds(0, _GB)]], rB, semG).wait()

    def fire_w(b, rA, rB, semW):
        off = pl.multiple_of(w * EPW + b * _GB, 16)
        if do_add:
            def add_row(r, _):
                va = [rA[r, pl.ds(v * L, L)] for v in range(d // L)]
                vb = [rB[r, pl.ds(v * L, L)] for v in range(d // L)]
                for v in range(d // L):
                    rA[r, pl.ds(v * L, L)] = va[v] + vb[v]
                return 0
            lax.fori_loop(0, _GB, add_row, 0)
            pltpu.async_copy(rA, oa_hbm.at[pl.ds(off, _GB)], semW)
        else:
            pltpu.async_copy(rA, oa_hbm.at[pl.ds(off, _GB)], semW)
            pltpu.async_copy(rB, ob_hbm.at[pl.ds(off, _GB)], semW)

    def wait_w(rA, rB, semW):
        pltpu.make_async_copy(rA, oa_hbm.at[pl.ds(0, _GB)], semW).wait()
        if not do_add:
            pltpu.make_async_copy(rB, ob_hbm.at[pl.ds(0, _GB)], semW).wait()

    stage(0, rA0, rB0, semG0)

    def pair(p, _):
        b0 = 2 * p
        b1 = 2 * p + 1

        @pl.when(b1 < _NGB)
        def _():
            @pl.when(p > 0)
            def _():
                wait_w(rA1, rB1, semW1)
            stage(b1, rA1, rB1, semG1)

        wait_g(rA0, rB0, semG0)
        fire_w(b0, rA0, rB0, semW0)

        @pl.when(b0 + 2 < _NGB)
        def _():
            wait_w(rA0, rB0, semW0)
            stage(b0 + 2, rA0, rB0, semG0)

        @pl.when(b1 < _NGB)
        def _():
            wait_g(rA1, rB1, semG1)
            fire_w(b1, rA1, rB1, semW1)
        return 0

    lax.fori_loop(0, (_NGB + 1) // 2, pair, 0)
    wait_w(rA0, rB0, semW0)
    if _NGB > 1:
        wait_w(rA1, rB1, semW1)


def _gather_scratch(d):
    return [pltpu.VMEM((EPW,), jnp.int32),
            pltpu.VMEM((EPW,), jnp.int32),
            pltpu.VMEM((_GB, d), jnp.float32),
            pltpu.VMEM((_GB, d), jnp.float32),
            pltpu.VMEM((_GB, d), jnp.float32),
            pltpu.VMEM((_GB, d), jnp.float32),
            pltpu.SemaphoreType.DMA,
            pltpu.SemaphoreType.DMA,
            pltpu.SemaphoreType.DMA,
            pltpu.SemaphoreType.DMA]


def _g2_body(table_hbm, dst_hbm, src_hbm, xi_hbm, xj_hbm, *rest):
    _gather_pipe_body(False, D, table_hbm, table_hbm, dst_hbm, src_hbm,
                      xi_hbm, xj_hbm, *rest)


_gather2 = functools.partial(
    pl.kernel, _g2_body,
    out_type=(jax.ShapeDtypeStruct((E, D), jnp.float32),
              jax.ShapeDtypeStruct((E, D), jnp.float32)),
    mesh=_MESH,
    compiler_params=pltpu.CompilerParams(needs_layout_passes=False),
    scratch_types=_gather_scratch(D),
)()


_D2 = 256


def _ga_body(ta_hbm, tb_hbm, dst_hbm, src_hbm, g_hbm, *rest):
    _gather_pipe_body(True, _D2, ta_hbm, tb_hbm, dst_hbm, src_hbm,
                      g_hbm, g_hbm, *rest)


_gather_add = functools.partial(
    pl.kernel, _ga_body,
    out_type=jax.ShapeDtypeStruct((E, _D2), jnp.float32),
    mesh=_MESH,
    compiler_params=pltpu.CompilerParams(needs_layout_passes=False),
    scratch_types=_gather_scratch(_D2),
)()


_SGB = {256: 128, 128: 128}


_WCH = 2048


def _segmax_body(dm, packed, m_hbm, lists_hbm, counts_hbm, h_hbm,
                 wbig, idb0, idb1, slb0, slb1, rows0, rows1,
                 acc, cnt_v, sem0, sem1):
    w = _wid()
    bg = _SGB[dm]
    wpb = _WCH // bg
    dw = dm // 2 if packed else dm
    zero = jnp.zeros((L,), jnp.int32 if packed else jnp.float32)

    def init_f(r, _):
        acc[pl.ds(r * L, L)] = zero
        return 0
    lax.fori_loop(0, NPW * dw // L, init_f, 0)

    pltpu.sync_copy(counts_hbm.at[pl.ds(pl.multiple_of(w * 128, 128), 128)], cnt_v)
    cnt = cnt_v[pl.ds(0, L)][0]
    nb = cnt // bg

    def stage(b, idb, slb, rows, sem):
        @pl.when(b % wpb == 0)
        def _():
            pltpu.sync_copy(
                lists_hbm.at[pl.ds(pl.multiple_of(w * ECAP, 128)
                                   + (b // wpb) * _WCH, _WCH)],
                wbig)
        rel = (b % wpb) * bg
        for j8 in range(bg // L):
            s = pl.ds(j8 * L, L)
            wv = wbig[pl.ds(rel + j8 * L, L)]
            idb[s] = wv >> SLOT_BITS
            slb[s] = wv & ((1 << SLOT_BITS) - 1)
        pltpu.async_copy(m_hbm.at[idb], rows, sem)

    def consume(slb, rows):
        def upd(j8, _):
            sv = slb[pl.ds(j8 * L, L)] * dw
            bases = [sv[jj] for jj in range(L)]
            for jj in range(L):
                j = j8 * L + jj
                base = bases[jj]
                rvals = [rows[j, pl.ds(v * L, L)] for v in range(dw // L)]
                avals = [acc[pl.ds(base + v * L, L)] for v in range(dw // L)]
                for v in range(dw // L):
                    if packed:
                        mx = jnp.maximum(
                            plsc.bitcast(avals[v], jnp.bfloat16),
                            plsc.bitcast(rvals[v], jnp.bfloat16))
                        acc[pl.ds(base + v * L, L)] = plsc.bitcast(
                            mx, jnp.int32)
                    else:
                        acc[pl.ds(base + v * L, L)] = jnp.maximum(
                            avals[v], rvals[v])
            return 0
        lax.fori_loop(0, bg // L, upd, 0)

    @pl.when(nb > 0)
    def _():
        stage(0, idb0, slb0, rows0, sem0)

    def pair(p, _):
        b0 = 2 * p
        b1 = 2 * p + 1

        @pl.when(b1 < nb)
        def _():
            stage(b1, idb1, slb1, rows1, sem1)
        pltpu.make_async_copy(m_hbm.at[idb0], rows0, sem0).wait()
        consume(slb0, rows0)

        @pl.when(b0 + 2 < nb)
        def _():
            stage(b0 + 2, idb0, slb0, rows0, sem0)

        @pl.when(b1 < nb)
        def _():
            pltpu.make_async_copy(m_hbm.at[idb1], rows1, sem1).wait()
            consume(slb1, rows1)
        return 0

    lax.fori_loop(0, (nb + 1) // 2, pair, 0)
    pltpu.sync_copy(acc, h_hbm.at[pl.ds(pl.multiple_of(w * NPW * dw, 128), NPW * dw)])


def _make_segmax(dm, packed):
    bg = _SGB[dm]
    dw = dm // 2 if packed else dm
    dt = jnp.int32 if packed else jnp.float32
    return functools.partial(
        pl.kernel, functools.partial(_segmax_body, dm, packed),
        out_type=jax.ShapeDtypeStruct((NPAD * dw,), dt),
        mesh=_MESH,
        compiler_params=pltpu.CompilerParams(needs_layout_passes=False),
        scratch_types=[pltpu.VMEM((_WCH,), jnp.int32),
                       pltpu.VMEM((bg,), jnp.int32),
                       pltpu.VMEM((bg,), jnp.int32),
                       pltpu.VMEM((bg,), jnp.int32),
                       pltpu.VMEM((bg,), jnp.int32),
                       pltpu.VMEM((bg, dw), dt),
                       pltpu.VMEM((bg, dw), dt),
                       pltpu.VMEM((NPW * dw,), dt),
                       pltpu.VMEM((128,), jnp.int32),
                       pltpu.SemaphoreType.DMA,
                       pltpu.SemaphoreType.DMA],
    )()


_segmax_256 = _make_segmax(256, True)
_segmax_128 = _make_segmax(128, False)


_BE = 512


def _mlp1_block(xi, xj, ea, wd, wm, we, b1a, w1b, b1b, out):
    h = (jnp.dot(xi[...].astype(jnp.bfloat16), wd[...],
                 preferred_element_type=jnp.float32)
         + jnp.dot(xj[...].astype(jnp.bfloat16), wm[...],
                   preferred_element_type=jnp.float32)
         + jnp.dot(ea[...], we[...], preferred_element_type=jnp.float32)
         + b1a[...])
    h = jnp.maximum(h, 0.0).astype(jnp.bfloat16)
    h2 = jnp.dot(h, w1b[...], preferred_element_type=jnp.float32) + b1b[...]
    h2 = jnp.maximum(h2, 0.0).astype(jnp.bfloat16)
    lo = jax.lax.bitcast_convert_type(h2[:, 0:128], jnp.uint16)
    hi = jax.lax.bitcast_convert_type(h2[:, 128:256], jnp.uint16)
    out[...] = (lo.astype(jnp.int32)
                | (hi.astype(jnp.int32) << 16))


def _mlp1(xi, xj, ea, wd, wm, we, b1a, w1b, b1b):
    grid = (E // _BE,)
    return pl.pallas_call(
        _mlp1_block,
        grid=grid,
        in_specs=[
            pl.BlockSpec((_BE, D), lambda i: (i, 0)),
            pl.BlockSpec((_BE, D), lambda i: (i, 0)),
            pl.BlockSpec((_BE, DE), lambda i: (i, 0)),
            pl.BlockSpec((D, 512), lambda i: (0, 0)),
            pl.BlockSpec((D, 512), lambda i: (0, 0)),
            pl.BlockSpec((DE, 512), lambda i: (0, 0)),
            pl.BlockSpec((1, 512), lambda i: (0, 0)),
            pl.BlockSpec((512, 256), lambda i: (0, 0)),
            pl.BlockSpec((1, 256), lambda i: (0, 0)),
        ],
        out_specs=pl.BlockSpec((_BE, 128), lambda i: (i, 0)),
        out_shape=jax.ShapeDtypeStruct((E, 128), jnp.int32),
    )(xi, xj, ea, wd, wm, we, b1a, w1b, b1b)


def _pre2_block(h1, wd, wm, a2, b2):
    wp = h1[...]
    lo = jax.lax.bitcast_convert_type(
        (wp & 0xFFFF).astype(jnp.uint16), jnp.bfloat16)
    hi = jax.lax.bitcast_convert_type(
        jax.lax.shift_right_logical(wp, 16).astype(jnp.uint16), jnp.bfloat16)
    hb = jnp.concatenate([lo, hi], axis=1)
    a2[...] = jnp.dot(hb, wd[...], preferred_element_type=jnp.float32)
    b2[...] = jnp.dot(hb, wm[...], preferred_element_type=jnp.float32)


def _pre2(h1, wd, wm):
    bn = NPAD // 4
    return pl.pallas_call(
        _pre2_block,
        grid=(4,),
        in_specs=[pl.BlockSpec((bn, _D2 // 2), lambda i: (i, 0)),
                  pl.BlockSpec((_D2, _D2), lambda i: (0, 0)),
                  pl.BlockSpec((_D2, _D2), lambda i: (0, 0))],
        out_specs=(pl.BlockSpec((bn, _D2), lambda i: (i, 0)),
                   pl.BlockSpec((bn, _D2), lambda i: (i, 0))),
        out_shape=(jax.ShapeDtypeStruct((NPAD, _D2), jnp.float32),
                   jax.ShapeDtypeStruct((NPAD, _D2), jnp.float32)),
    )(h1, wd, wm)


def _mlp2_block(g2, ea, we, b2a, w2b, b2b, out):
    h = (g2[...]
         + jnp.dot(ea[...], we[...], preferred_element_type=jnp.float32)
         + b2a[...])
    h = jnp.maximum(h, 0.0).astype(jnp.bfloat16)
    h2 = jnp.dot(h, w2b[...], preferred_element_type=jnp.float32) + b2b[...]
    out[...] = jnp.maximum(h2, 0.0)


def _mlp2(g2, ea, we, b2a, w2b, b2b):
    return pl.pallas_call(
        _mlp2_block,
        grid=(E // _BE,),
        in_specs=[
            pl.BlockSpec((_BE, _D2), lambda i: (i, 0)),
            pl.BlockSpec((_BE, DE), lambda i: (i, 0)),
            pl.BlockSpec((DE, _D2), lambda i: (0, 0)),
            pl.BlockSpec((1, _D2), lambda i: (0, 0)),
            pl.BlockSpec((_D2, D), lambda i: (0, 0)),
            pl.BlockSpec((1, D), lambda i: (0, 0)),
        ],
        out_specs=pl.BlockSpec((_BE, D), lambda i: (i, 0)),
        out_shape=jax.ShapeDtypeStruct((E, D), jnp.float32),
    )(g2, ea, we, b2a, w2b, b2b)


def _final_block(h2, w3, b3, w4, b4, out):
    h = jnp.dot(h2[...], w3[...], preferred_element_type=jnp.float32) + b3[...]
    h = jnp.maximum(h, 0.0)
    o = jnp.dot(h, w4[...], preferred_element_type=jnp.float32) + b4[...]
    out[...] = jax.nn.sigmoid(o)


def _final(h2, w3, b3, w4, b4):
    bn = NPAD // 4
    return pl.pallas_call(
        _final_block,
        grid=(4,),
        in_specs=[
            pl.BlockSpec((bn, D), lambda i: (i, 0)),
            pl.BlockSpec((D, 64), lambda i: (0, 0)),
            pl.BlockSpec((1, 64), lambda i: (0, 0)),
            pl.BlockSpec((64, 1), lambda i: (0, 0)),
            pl.BlockSpec((1, 1), lambda i: (0, 0)),
        ],
        out_specs=pl.BlockSpec((bn, 1), lambda i: (i, 0)),
        out_shape=jax.ShapeDtypeStruct((NPAD, 1), jnp.float32),
    )(h2, w3, b3, w4, b4)


def kernel(x, edge_index, edge_attr, W1a, b1a, W1b, b1b, W2a, b2a, W2b, b2b,
           W3, b3, W4, b4):
    src = edge_index[0].astype(jnp.int32)
    dst = edge_index[1].astype(jnp.int32)
    bf = jnp.bfloat16

    lists, counts = _bucketize(dst)

    xi, xj = _gather2(x, dst, src)
    w1d = (W1a[0:D] - W1a[D:2 * D]).astype(bf)
    m1 = _mlp1(xi, xj, edge_attr.astype(bf),
               w1d, W1a[D:2 * D].astype(bf), W1a[2 * D:].astype(bf),
               b1a.reshape(1, -1), W1b.astype(bf), b1b.reshape(1, -1))
    h1 = _segmax_256(m1, lists, counts).reshape(NPAD, _D2 // 2)

    w2d = (W2a[0:_D2] - W2a[_D2:2 * _D2]).astype(bf)
    a2, b2t = _pre2(h1, w2d, W2a[_D2:2 * _D2].astype(bf))
    g2 = _gather_add(a2, b2t, dst, src)
    m2 = _mlp2(g2, edge_attr.astype(bf),
               W2a[2 * _D2:].astype(bf), b2a.reshape(1, -1),
               W2b.astype(bf), b2b.reshape(1, -1))
    h2 = _segmax_128(m2, lists, counts).reshape(NPAD, D)

    out = _final(h2, W3, b3.reshape(1, -1), W4, b4.reshape(1, -1))
    return out[:N]
